# baseline (speedup 1.0000x reference)
import functools

import jax
import jax.numpy as jnp
from jax import lax
from jax.experimental import pallas as pl
from jax.experimental.pallas import tpu as pltpu
from jax.experimental.pallas import tpu_sc as plsc

EMB_DIM = 64
NUM_LABELS = 16
BATCH = 4096
SEQ_LEN = 200
CH = SEQ_LEN // 2


def _pool_sc(ids2, table):
    info = plsc.get_sparse_core_info()
    nc, ns = info.num_cores, info.num_subcores
    nw = nc * ns
    bpw = BATCH // nw
    mesh = plsc.VectorSubcoreMesh(core_axis_name="c", subcore_axis_name="s")

    @functools.partial(
        pl.kernel,
        mesh=mesh,
        out_type=jax.ShapeDtypeStruct((BATCH, EMB_DIM), jnp.float32),
        compiler_params=pltpu.CompilerParams(use_tc_tiling_on_sc=False),
        scratch_types=[
            pltpu.VMEM((2 * bpw, CH), jnp.int32),
            pltpu.VMEM((CH, EMB_DIM), jnp.float32),
            pltpu.VMEM((CH, EMB_DIM), jnp.float32),
            pltpu.VMEM((bpw, EMB_DIM), jnp.float32),
            pltpu.SemaphoreType.DMA,
            pltpu.SemaphoreType.DMA,
        ],
    )
    def k(ids_hbm, table_hbm, out_hbm, idx_v, buf0, buf1, out_v, sem0, sem1):
        wid = lax.axis_index("s") * nc + lax.axis_index("c")
        pltpu.sync_copy(ids_hbm.at[pl.ds(wid * (2 * bpw), 2 * bpw)], idx_v)

        def start(chunk, buf, sem):
            pltpu.async_copy(table_hbm.at[idx_v.at[chunk]], buf, sem)

        def wait(chunk, buf, sem):
            pltpu.make_async_copy(table_hbm.at[idx_v.at[chunk]], buf, sem).wait()

        def reduce_max(buf, acc):
            def jbody(j, acc):
                a0, a1, a2, a3 = acc
                a0 = jnp.maximum(a0, buf[j, pl.ds(0, 16)])
                a1 = jnp.maximum(a1, buf[j, pl.ds(16, 16)])
                a2 = jnp.maximum(a2, buf[j, pl.ds(32, 16)])
                a3 = jnp.maximum(a3, buf[j, pl.ds(48, 16)])
                return (a0, a1, a2, a3)

            return lax.fori_loop(0, CH, jbody, acc)

        start(0, buf0, sem0)

        def body(b, _):
            start(2 * b + 1, buf1, sem1)
            wait(2 * b, buf0, sem0)
            ninf = jnp.full((16,), -jnp.inf, jnp.float32)
            acc = reduce_max(buf0, (ninf, ninf, ninf, ninf))

            @pl.when(b + 1 < bpw)
            def _():
                start(2 * b + 2, buf0, sem0)

            wait(2 * b + 1, buf1, sem1)
            a0, a1, a2, a3 = reduce_max(buf1, acc)
            out_v[b, pl.ds(0, 16)] = a0
            out_v[b, pl.ds(16, 16)] = a1
            out_v[b, pl.ds(32, 16)] = a2
            out_v[b, pl.ds(48, 16)] = a3
            return 0

        lax.fori_loop(0, bpw, body, 0)
        pltpu.sync_copy(out_v, out_hbm.at[pl.ds(wid * bpw, bpw)])

    return k(ids2, table)


def _head_tc(pooled, W, b2):
    def mm(x_ref, w_ref, b_ref, o_ref):
        o_ref[...] = (
            jnp.dot(x_ref[...], w_ref[...], preferred_element_type=jnp.float32)
            + b_ref[...]
        )

    return pl.pallas_call(
        mm,
        out_shape=jax.ShapeDtypeStruct((BATCH, NUM_LABELS), jnp.float32),
    )(pooled, W, b2)


def kernel(padded_token_ids, lengths, emb_table, W, b):
    del lengths
    ids2 = padded_token_ids.astype(jnp.int32).reshape(2 * BATCH, CH)
    pooled = _pool_sc(ids2, emb_table)
    return _head_tc(pooled, W.astype(jnp.float32), b.reshape(1, NUM_LABELS))


# speedup vs baseline: 2.1909x; 2.1909x over previous
import functools

import jax
import jax.numpy as jnp
from jax import lax
from jax.experimental import pallas as pl
from jax.experimental.pallas import tpu as pltpu
from jax.experimental.pallas import tpu_sc as plsc

EMB_DIM = 64
NUM_LABELS = 16
BATCH = 4096
SEQ_LEN = 200
CH = SEQ_LEN // 2
BLK = 12288
NSUB = 4


def _encode16(t):
    s = t >> jnp.uint32(15)
    mask = ((jnp.uint32(0) - s) & jnp.uint32(0x7FFF)) | jnp.uint32(0x8000)
    return t ^ mask


def _pack_tc(table_t):
    v = table_t.shape[1]
    grid = (v + NSUB * BLK - 1) // (NSUB * BLK)

    def body(in_ref, o_ref):
        for q in range(NSUB):
            u = lax.bitcast_convert_type(
                in_ref[:, pl.ds(q * BLK, BLK)], jnp.uint32
            )
            u = u + jnp.uint32(0x8000)
            hi = _encode16(u[:32, :] >> jnp.uint32(16))
            lo = _encode16(u[32:, :] >> jnp.uint32(16))
            w = (hi << jnp.uint32(16)) | lo
            o_ref[:, pl.ds(q * 32, 32)] = w.T

    return pl.pallas_call(
        body,
        grid=(grid,),
        in_specs=[pl.BlockSpec((EMB_DIM, NSUB * BLK), lambda i: (0, i))],
        out_specs=pl.BlockSpec((BLK, 2 * EMB_DIM), lambda i: (i, 0)),
        out_shape=jax.ShapeDtypeStruct((grid * BLK, 2 * EMB_DIM), jnp.uint32),
    )(table_t)


def _remap_ids(ids):
    i = ids // (NSUB * BLK)
    u = ids % BLK
    q = (ids // BLK) % NSUB
    return NSUB * (i * BLK + u) + q


def _pool_sc(ids2, packed):
    info = plsc.get_sparse_core_info()
    nc, ns = info.num_cores, info.num_subcores
    nw = nc * ns
    bpw = BATCH // nw
    mesh = plsc.VectorSubcoreMesh(core_axis_name="c", subcore_axis_name="s")

    @functools.partial(
        pl.kernel,
        mesh=mesh,
        out_type=jax.ShapeDtypeStruct((BATCH, 32), jnp.uint32),
        compiler_params=pltpu.CompilerParams(use_tc_tiling_on_sc=False),
        scratch_types=[
            pltpu.VMEM((2 * bpw, CH), jnp.int32),
            pltpu.VMEM((CH, 32), jnp.uint32),
            pltpu.VMEM((CH, 32), jnp.uint32),
            pltpu.VMEM((CH, 32), jnp.uint32),
            pltpu.VMEM((CH, 32), jnp.uint32),
            pltpu.VMEM((bpw, 32), jnp.uint32),
            pltpu.SemaphoreType.DMA,
            pltpu.SemaphoreType.DMA,
            pltpu.SemaphoreType.DMA,
            pltpu.SemaphoreType.DMA,
        ],
    )
    def k(ids_hbm, tbl_hbm, out_hbm, idx_v, buf0, buf1, buf2, buf3, out_v,
          sem0, sem1, sem2, sem3):
        wid = lax.axis_index("s") * nc + lax.axis_index("c")
        pltpu.sync_copy(ids_hbm.at[pl.ds(wid * (2 * bpw), 2 * bpw)], idx_v)

        bufs = (buf0, buf1, buf2, buf3)
        sems = (sem0, sem1, sem2, sem3)
        nchunks = 2 * bpw
        hmask = jnp.full((16,), 0xFFFF0000, jnp.uint32)
        lmask = jnp.full((16,), 0x0000FFFF, jnp.uint32)

        def start(chunk, kslot):
            pltpu.async_copy(tbl_hbm.at[idx_v.at[chunk]], bufs[kslot], sems[kslot])

        def wait(chunk, kslot):
            pltpu.make_async_copy(
                tbl_hbm.at[idx_v.at[chunk]], bufs[kslot], sems[kslot]
            ).wait()

        def reduce_max(buf, acc):
            def jbody(j, acc):
                h0, l0, h1, l1 = acc
                w0 = buf[j, pl.ds(0, 16)]
                w1 = buf[j, pl.ds(16, 16)]
                h0 = jnp.maximum(h0, w0 & hmask)
                l0 = jnp.maximum(l0, w0 & lmask)
                h1 = jnp.maximum(h1, w1 & hmask)
                l1 = jnp.maximum(l1, w1 & lmask)
                return (h0, l0, h1, l1)

            return lax.fori_loop(0, CH, jbody, acc, unroll=4)

        for c in range(4):
            start(c, c)

        def group(g, _):
            z = jnp.zeros((16,), jnp.uint32)
            c0 = 4 * g
            acc = (z, z, z, z)
            for kslot in range(4):
                wait(c0 + kslot, kslot)
                acc = reduce_max(bufs[kslot], (z, z, z, z) if kslot == 2 else acc)
                if kslot == 1:
                    h0, l0, h1, l1 = acc
                    out_v[2 * g, pl.ds(0, 16)] = h0 | l0
                    out_v[2 * g, pl.ds(16, 16)] = h1 | l1

                @pl.when(c0 + kslot + 4 < nchunks)
                def _(c=c0 + kslot + 4, kslot=kslot):
                    start(c, kslot)

            h0, l0, h1, l1 = acc
            out_v[2 * g + 1, pl.ds(0, 16)] = h0 | l0
            out_v[2 * g + 1, pl.ds(16, 16)] = h1 | l1
            return 0

        lax.fori_loop(0, bpw // 2, group, 0)
        pltpu.sync_copy(out_v, out_hbm.at[pl.ds(wid * bpw, bpw)])

    return k(ids2, packed)


def _decode(pooled_u):
    e = jnp.concatenate(
        [pooled_u >> jnp.uint32(16), pooled_u & jnp.uint32(0xFFFF)], axis=1
    )
    s = e >> jnp.uint32(15)
    mask = jnp.where(s == 1, jnp.uint32(0x8000), jnp.uint32(0xFFFF))
    bits = (e ^ mask) << jnp.uint32(16)
    return lax.bitcast_convert_type(bits, jnp.float32)


def _head_tc(pooled, W, b2):
    def mm(x_ref, w_ref, b_ref, o_ref):
        o_ref[...] = (
            jnp.dot(x_ref[...], w_ref[...], preferred_element_type=jnp.float32)
            + b_ref[...]
        )

    return pl.pallas_call(
        mm,
        out_shape=jax.ShapeDtypeStruct((BATCH, NUM_LABELS), jnp.float32),
    )(pooled, W, b2)


def kernel(padded_token_ids, lengths, emb_table, W, b):
    del lengths
    ids = padded_token_ids.astype(jnp.int32)
    ids2 = _remap_ids(ids).reshape(2 * BATCH, CH)

    packed2d = _pack_tc(emb_table.T)
    packed = packed2d.reshape(packed2d.shape[0] * NSUB, 32)

    pooled_u = _pool_sc(ids2, packed)
    pooled = _decode(pooled_u)
    return _head_tc(pooled, W.astype(jnp.float32), b.reshape(1, NUM_LABELS))
